# baseline (speedup 1.0000x reference)
import functools

import jax
import jax.numpy as jnp
from jax import lax
from jax.experimental import pallas as pl
from jax.experimental.pallas import tpu as pltpu
from jax.experimental.pallas import tpu_sc as plsc

_B = 1024
_S = 50
_K = 100000
_TOPK = 30
_D = 128
_EPS = 1e-12

_ROWS = 8
_NEG = -3.0e38


def _topk_body(x_ref, lab_ref, s_ref, col_ref):
    s_ref[...] = x_ref[...]
    col_ref[...] = lax.broadcasted_iota(jnp.int32, (_ROWS, _K), 1)
    lane32 = lax.broadcasted_iota(jnp.int32, (_ROWS, 32), 1)

    def body(t, lab):
        s = s_ref[...]
        col = col_ref[...]
        m = jnp.max(s, axis=1, keepdims=True)
        eq = s == m
        idx = jnp.min(jnp.where(eq, col, _K), axis=1, keepdims=True)
        s_ref[...] = jnp.where(col == idx, _NEG, s)
        return jnp.where(lane32 == t, idx, lab)

    lab = lax.fori_loop(0, _TOPK, body, jnp.zeros((_ROWS, 32), jnp.int32))
    lab_ref[...] = lab


def _topk(preds_attr):
    grid = _B // _ROWS
    lab = pl.pallas_call(
        _topk_body,
        grid=(grid,),
        in_specs=[pl.BlockSpec((_ROWS, _K), lambda i: (i, 0))],
        out_specs=pl.BlockSpec((_ROWS, 32), lambda i: (i, 0)),
        out_shape=jax.ShapeDtypeStruct((_B, 32), jnp.int32),
        scratch_shapes=[
            pltpu.VMEM((_ROWS, _K), jnp.float32),
            pltpu.VMEM((_ROWS, _K), jnp.int32),
        ],
        compiler_params=pltpu.CompilerParams(
            dimension_semantics=("arbitrary",),
        ),
    )(preds_attr)
    return lab[:, :_TOPK]


def _make_sc_gather():
    nc, ns = 2, 16
    nw = nc * ns
    n = _B * _TOPK
    b_per_w = n // nw
    mesh = plsc.VectorSubcoreMesh(core_axis_name="c", subcore_axis_name="s")

    @functools.partial(
        pl.kernel,
        mesh=mesh,
        out_type=jax.ShapeDtypeStruct((n, _D), jnp.float32),
        scratch_types=[
            pltpu.VMEM((b_per_w,), jnp.int32),
            pltpu.VMEM((b_per_w, _D), jnp.float32),
            pltpu.SemaphoreType.DMA,
        ],
    )
    def gather_k(table_hbm, idx_hbm, out_hbm, idx_v, rows_v, sem):
        wid = lax.axis_index("s") * nc + lax.axis_index("c")
        base = wid * b_per_w
        pltpu.sync_copy(idx_hbm.at[pl.ds(base, b_per_w)], idx_v)
        pltpu.async_copy(table_hbm.at[idx_v], rows_v, sem).wait()
        pltpu.sync_copy(rows_v, out_hbm.at[pl.ds(base, b_per_w)])

    return gather_k


def _ln_body(x_ref, pos_ref, g_ref, b_ref, o_ref):
    x = x_ref[...]
    pos = jnp.tile(pos_ref[...], (x.shape[0] // _TOPK, 1))
    y = x + pos
    mu = jnp.mean(y, axis=1, keepdims=True)
    d = y - mu
    var = jnp.mean(d * d, axis=1, keepdims=True)
    o_ref[...] = d / jnp.sqrt(var + _EPS) * g_ref[...] + b_ref[...]


def _ln(embs_flat, pos_emb, ln_gamma, ln_beta):
    rows = 240
    grid = (_B * _TOPK) // rows
    return pl.pallas_call(
        _ln_body,
        grid=(grid,),
        in_specs=[
            pl.BlockSpec((rows, _D), lambda i: (i, 0)),
            pl.BlockSpec((_TOPK, _D), lambda i: (0, 0)),
            pl.BlockSpec((1, _D), lambda i: (0, 0)),
            pl.BlockSpec((1, _D), lambda i: (0, 0)),
        ],
        out_specs=pl.BlockSpec((rows, _D), lambda i: (i, 0)),
        out_shape=jax.ShapeDtypeStruct((_B * _TOPK, _D), jnp.float32),
        compiler_params=pltpu.CompilerParams(
            dimension_semantics=("arbitrary",),
        ),
    )(embs_flat, pos_emb, ln_gamma, ln_beta)


def kernel(encoder_hidden_states, preds_attr, word_emb, pos_emb, ln_gamma, ln_beta):
    labels = _topk(preds_attr)
    idx_flat = labels.reshape(_B * _TOPK)
    embs_flat = _make_sc_gather()(word_emb, idx_flat)
    out = _ln(
        embs_flat,
        pos_emb,
        ln_gamma.reshape(1, _D),
        ln_beta.reshape(1, _D),
    )
    return out.reshape(_B, _TOPK, _D), labels
